# Initial kernel scaffold; baseline (speedup 1.0000x reference)
import jax
import jax.numpy as jnp
from jax.experimental import pallas as pl


def kernel(node_fts, edge_index, edge_weight, W, b):
    raise NotImplementedError("write your pallas kernel here")


# speedup vs baseline: 8.8176x; 8.8176x over previous
import functools

import jax
import jax.numpy as jnp
from jax import lax
from jax.experimental import pallas as pl
from jax.experimental.pallas import tpu as pltpu
from jax.experimental.pallas import tpu_sc as plsc

N_NODES = 10000
N_EDGES = 320000
FT = 128

NC = 2
NS = 16
NW = NC * NS

CHUNK = 320
HALF = CHUNK // 2
N_CHUNKS = N_EDGES // CHUNK
GROUPS = HALF // 16
ROWS_PER_TILE = 624
ROWS_REM = N_NODES - NS * ROWS_PER_TILE
VREGS_PER_ROW = FT // 16


def _sc_agg_body(nf_hbm, src_hbm, dst_hbm, w_hbm, out_hbm,
                 src_a, src_b, dst_a, dst_b, w_a, w_b, rows_a, rows_b,
                 acc, gsem_a, gsem_b, ssem_a, ssem_b):
    c = lax.axis_index("c")
    s = lax.axis_index("s")
    wid = s * NC + c

    zero = jnp.zeros((16,), jnp.float32)

    def _zero_row(e, _):
        for r in range(VREGS_PER_ROW):
            rows_a[e, pl.ds(r * 16, 16)] = zero
        return 0

    lax.fori_loop(0, HALF, _zero_row, 0)
    row0 = s * ROWS_PER_TILE
    for j in range(ROWS_PER_TILE // HALF):
        pltpu.sync_copy(rows_a, acc.at[pl.ds(row0 + j * HALF, HALF)])
    _rem = ROWS_PER_TILE % HALF
    if _rem:
        pltpu.sync_copy(rows_a.at[pl.ds(0, _rem)],
                        acc.at[pl.ds(row0 + ROWS_PER_TILE - _rem, _rem)])

    @pl.when(s == 0)
    def _zero_tail():
        pltpu.sync_copy(rows_a.at[pl.ds(0, ROWS_REM)],
                        acc.at[pl.ds(NS * ROWS_PER_TILE, ROWS_REM)])

    plsc.subcore_barrier()

    my_chunks = (N_CHUNKS - wid + NW - 1) // NW

    def _load_idx(k, half, src_v, dst_v, w_v):
        off = (wid + k * NW) * CHUNK + half * HALF
        pltpu.sync_copy(src_hbm.at[pl.ds(off, HALF)], src_v)
        pltpu.sync_copy(dst_hbm.at[pl.ds(off, HALF)], dst_v)
        pltpu.sync_copy(w_hbm.at[pl.ds(off, HALF)], w_v)

    def _scale(rows_v, w_v):
        def _scale_group(g, _):
            wv = w_v[pl.ds(g * 16, 16)]
            for i in range(16):
                e = g * 16 + i
                w = wv[i]
                for r in range(VREGS_PER_ROW):
                    sl = pl.ds(r * 16, 16)
                    rows_v[e, sl] = rows_v[e, sl] * w
            return 0

        lax.fori_loop(0, GROUPS, _scale_group, 0)

    _load_idx(0, 0, src_a, dst_a, w_a)
    pltpu.async_copy(nf_hbm.at[src_a], rows_a, gsem_a)
    _load_idx(0, 1, src_b, dst_b, w_b)
    pltpu.async_copy(nf_hbm.at[src_b], rows_b, gsem_b)

    def _chunk(k, _):
        pltpu.make_async_copy(nf_hbm.at[src_a], rows_a, gsem_a).wait()
        _scale(rows_a, w_a)
        pltpu.async_copy(rows_a, acc.at[dst_a], ssem_a, add=True)
        pltpu.make_async_copy(nf_hbm.at[src_b], rows_b, gsem_b).wait()
        _scale(rows_b, w_b)
        pltpu.async_copy(rows_b, acc.at[dst_b], ssem_b, add=True)

        @pl.when(k + 1 < my_chunks)
        def _prefetch():
            pltpu.make_async_copy(rows_a, acc.at[dst_a], ssem_a).wait()
            _load_idx(k + 1, 0, src_a, dst_a, w_a)
            pltpu.async_copy(nf_hbm.at[src_a], rows_a, gsem_a)
            pltpu.make_async_copy(rows_b, acc.at[dst_b], ssem_b).wait()
            _load_idx(k + 1, 1, src_b, dst_b, w_b)
            pltpu.async_copy(nf_hbm.at[src_b], rows_b, gsem_b)

        return 0

    lax.fori_loop(0, my_chunks, _chunk, 0)
    pltpu.make_async_copy(rows_a, acc.at[dst_a], ssem_a).wait()
    pltpu.make_async_copy(rows_b, acc.at[dst_b], ssem_b).wait()
    plsc.subcore_barrier()

    pltpu.sync_copy(acc.at[pl.ds(row0, ROWS_PER_TILE)],
                    out_hbm.at[c, pl.ds(row0, ROWS_PER_TILE)])

    @pl.when(s == 0)
    def _write_tail():
        pltpu.sync_copy(acc.at[pl.ds(NS * ROWS_PER_TILE, ROWS_REM)],
                        out_hbm.at[c, pl.ds(NS * ROWS_PER_TILE, ROWS_REM)])


@functools.partial(
    pl.kernel,
    out_type=jax.ShapeDtypeStruct((NC, N_NODES, FT), jnp.float32),
    mesh=plsc.VectorSubcoreMesh(core_axis_name="c", subcore_axis_name="s"),
    scratch_types=[
        pltpu.VMEM((HALF,), jnp.int32),
        pltpu.VMEM((HALF,), jnp.int32),
        pltpu.VMEM((HALF,), jnp.int32),
        pltpu.VMEM((HALF,), jnp.int32),
        pltpu.VMEM((HALF,), jnp.float32),
        pltpu.VMEM((HALF,), jnp.float32),
        pltpu.VMEM((HALF, FT), jnp.float32),
        pltpu.VMEM((HALF, FT), jnp.float32),
        pltpu.VMEM_SHARED((N_NODES, FT), jnp.float32),
        pltpu.SemaphoreType.DMA,
        pltpu.SemaphoreType.DMA,
        pltpu.SemaphoreType.DMA,
        pltpu.SemaphoreType.DMA,
    ],
)
def _sc_agg(*args):
    _sc_agg_body(*args)


ROW_BLK = 1000


def _tc_post_body(p_ref, w_ref, b_ref, o_ref):
    p = p_ref[0] + p_ref[1]
    y = lax.dot_general(p, w_ref[...], (((1,), (1,)), ((), ())),
                        preferred_element_type=jnp.float32)
    o_ref[...] = jnp.maximum(y + b_ref[...], 0.0)


def _tc_post(partials, W, b2d):
    return pl.pallas_call(
        _tc_post_body,
        out_shape=jax.ShapeDtypeStruct((N_NODES, FT), jnp.float32),
        grid=(N_NODES // ROW_BLK,),
        in_specs=[
            pl.BlockSpec((NC, ROW_BLK, FT), lambda i: (0, i, 0)),
            pl.BlockSpec((FT, FT), lambda i: (0, 0)),
            pl.BlockSpec((1, FT), lambda i: (0, 0)),
        ],
        out_specs=pl.BlockSpec((ROW_BLK, FT), lambda i: (i, 0)),
    )(partials, W, b2d)


def kernel(node_fts, edge_index, edge_weight, W, b):
    src = edge_index[1].astype(jnp.int32)
    dst = edge_index[0].astype(jnp.int32)
    partials = _sc_agg(node_fts, src, dst, edge_weight)
    return _tc_post(partials, W, b.reshape(1, FT))
